# baseline (speedup 1.0000x reference)
import functools

import jax
import jax.numpy as jnp
from jax.experimental import pallas as pl
from jax.experimental.pallas import tpu as pltpu
from jax.experimental.pallas import tpu_sc as plsc


_GATHER_WINDOW = 128


def _sc_gather(table, word):
    batch, embed_dim = word.shape[0], table.shape[1]
    indices = word.reshape(1, batch)
    mesh = plsc.VectorSubcoreMesh(core_axis_name="core", subcore_axis_name="subcore")

    @pl.kernel(
        out_type=jax.ShapeDtypeStruct((batch, embed_dim), table.dtype),
        mesh=mesh,
    )
    def gather_kernel(table_hbm, idx_hbm, out_hbm):
        def body(idx_vmem, out_vmem):
            pltpu.sync_copy(table_hbm.at[idx_vmem.at[0]], out_vmem)

        pltpu.emit_pipeline(
            body,
            grid=(batch // _GATHER_WINDOW,),
            in_specs=[pl.BlockSpec((1, _GATHER_WINDOW), lambda i: (0, i))],
            out_specs=[pl.BlockSpec((_GATHER_WINDOW, embed_dim), lambda i: (i, 0))],
            core_axis_name="subcore",
            dimension_semantics=(pltpu.PARALLEL,),
        )(idx_hbm, out_hbm)

    return gather_kernel(table, indices)


_R_BLOCK = 1024
_N_CHUNK = 8


def _proj_softmax_t(embed, wt):
    batch = embed.shape[0]
    vocab = wt.shape[1]
    n_steps = pl.cdiv(vocab, _R_BLOCK)
    last = n_steps - 1
    tail = vocab - last * _R_BLOCK

    def chunk_slices(width):
        per = _R_BLOCK // _N_CHUNK
        out = []
        off = 0
        while off < width:
            size = min(per, width - off)
            out.append((off, size))
            off += size
        return out

    per = _R_BLOCK // _N_CHUNK

    def body(embed_ref, w_ref, out_hbm, obuf, sems):
        j = pl.program_id(0)

        def dmas(slot, jj, width):
            return [
                pltpu.make_async_copy(
                    obuf.at[slot, pl.ds(off, size), :],
                    out_hbm.at[pl.ds(jj * _R_BLOCK + off, size), :],
                    sems.at[slot, c],
                )
                for c, (off, size) in enumerate(chunk_slices(width))
            ]

        def run_block(slot, dma_width):
            wb = w_ref[...].astype(jnp.bfloat16)
            eb = embed_ref[...].astype(jnp.bfloat16)
            out_cps = dmas(slot, j, dma_width)
            for c in range(_N_CHUNK):
                off = c * per
                logits_t = jax.lax.dot_general(
                    wb[:, off:off + per], eb,
                    dimension_numbers=(((0,), (1,)), ((), ())),
                    preferred_element_type=jnp.float32,
                )
                row_max = jnp.max(logits_t, axis=1, keepdims=True)
                lse = jnp.log(
                    jnp.sum(jnp.exp(logits_t - row_max), axis=1, keepdims=True))
                obuf[slot, pl.ds(off, per), :] = logits_t - (row_max + lse)
                if c < len(out_cps):
                    out_cps[c].start()

        def step(slot):
            @pl.when(j >= 2)
            def _():
                for cp in dmas(slot, j - 2, _R_BLOCK):
                    cp.wait()

            @pl.when(j != last)
            def _():
                run_block(slot, _R_BLOCK)

            @pl.when(j == last)
            def _():
                run_block(slot, tail)
                for cp in dmas(slot, j, tail):
                    cp.wait()
                for cp in dmas(1 - slot, j - 1, _R_BLOCK):
                    cp.wait()

        @pl.when(j % 2 == 0)
        def _():
            step(0)

        @pl.when(j % 2 == 1)
        def _():
            step(1)

    return pl.pallas_call(
        body,
        grid=(n_steps,),
        in_specs=[
            pl.BlockSpec((batch, embed.shape[1]), lambda i: (0, 0)),
            pl.BlockSpec((embed.shape[1], _R_BLOCK), lambda i: (0, i)),
        ],
        out_specs=pl.BlockSpec(memory_space=pl.ANY),
        out_shape=jax.ShapeDtypeStruct((vocab, batch), jnp.float32),
        scratch_shapes=[
            pltpu.VMEM((2, _R_BLOCK, batch), jnp.float32),
            pltpu.SemaphoreType.DMA((2, _N_CHUNK)),
        ],
    )(embed, wt)


def kernel(word, table, W, b):
    del b
    word = word.astype(jnp.int32)
    vocab, embed_dim = table.shape
    table2 = jnp.concatenate([table[0::2], table[1::2]], axis=1)
    pair = _sc_gather(table2, word >> 1)
    embed = jnp.where(
        (word & 1)[:, None] == 1, pair[:, embed_dim:], pair[:, :embed_dim]
    )
    out_t = _proj_softmax_t(embed, W.T)
    return out_t.T


# speedup vs baseline: 3.4146x; 3.4146x over previous
import functools

import jax
import jax.numpy as jnp
from jax.experimental import pallas as pl
from jax.experimental.pallas import tpu as pltpu
from jax.experimental.pallas import tpu_sc as plsc


_GATHER_WINDOW = 128


def _sc_gather(table, word):
    batch, embed_dim = word.shape[0], table.shape[1]
    indices = word.reshape(1, batch)
    mesh = plsc.VectorSubcoreMesh(core_axis_name="core", subcore_axis_name="subcore")

    @pl.kernel(
        out_type=jax.ShapeDtypeStruct((batch, embed_dim), table.dtype),
        mesh=mesh,
    )
    def gather_kernel(table_hbm, idx_hbm, out_hbm):
        def body(idx_vmem, out_vmem):
            pltpu.sync_copy(table_hbm.at[idx_vmem.at[0]], out_vmem)

        pltpu.emit_pipeline(
            body,
            grid=(batch // _GATHER_WINDOW,),
            in_specs=[pl.BlockSpec((1, _GATHER_WINDOW), lambda i: (0, i))],
            out_specs=[pl.BlockSpec((_GATHER_WINDOW, embed_dim), lambda i: (i, 0))],
            core_axis_name="subcore",
            dimension_semantics=(pltpu.PARALLEL,),
        )(idx_hbm, out_hbm)

    return gather_kernel(table, indices)


_R_BLOCK = 512
_N_CHUNK = 4


def _proj_softmax_t(embed, wt):
    batch = embed.shape[0]
    vocab = wt.shape[1]
    n_steps = pl.cdiv(vocab, _R_BLOCK)
    last = n_steps - 1
    tail = vocab - last * _R_BLOCK

    def chunk_slices(width):
        per = _R_BLOCK // _N_CHUNK
        out = []
        off = 0
        while off < width:
            size = min(per, width - off)
            out.append((off, size))
            off += size
        return out

    per = _R_BLOCK // _N_CHUNK

    def body(embed_ref, w_ref, out_hbm, obuf, sems):
        j = pl.program_id(0)

        def dmas(slot, jj, width):
            return [
                pltpu.make_async_copy(
                    obuf.at[slot, pl.ds(off, size), :],
                    out_hbm.at[pl.ds(jj * _R_BLOCK + off, size), :],
                    sems.at[slot, c],
                )
                for c, (off, size) in enumerate(chunk_slices(width))
            ]

        def run_block(slot, dma_width):
            wb = w_ref[...].astype(jnp.bfloat16)
            eb = embed_ref[...].astype(jnp.bfloat16)
            out_cps = dmas(slot, j, dma_width)
            for c in range(_N_CHUNK):
                off = c * per
                logits_t = jax.lax.dot_general(
                    wb[:, off:off + per], eb,
                    dimension_numbers=(((0,), (1,)), ((), ())),
                    preferred_element_type=jnp.float32,
                )
                row_max = jnp.max(logits_t, axis=1, keepdims=True)
                lse = jnp.log(
                    jnp.sum(jnp.exp(logits_t - row_max), axis=1, keepdims=True))
                obuf[slot, pl.ds(off, per), :] = logits_t - (row_max + lse)
                if c < len(out_cps):
                    out_cps[c].start()

        def step(slot):
            @pl.when(j >= 2)
            def _():
                for cp in dmas(slot, j - 2, _R_BLOCK):
                    cp.wait()

            @pl.when(j != last)
            def _():
                run_block(slot, _R_BLOCK)

            @pl.when(j == last)
            def _():
                run_block(slot, tail)
                for cp in dmas(slot, j, tail):
                    cp.wait()
                for cp in dmas(1 - slot, j - 1, _R_BLOCK):
                    cp.wait()

        @pl.when(j % 2 == 0)
        def _():
            step(0)

        @pl.when(j % 2 == 1)
        def _():
            step(1)

    return pl.pallas_call(
        body,
        grid=(n_steps,),
        in_specs=[
            pl.BlockSpec((batch, embed.shape[1]), lambda i: (0, 0)),
            pl.BlockSpec((embed.shape[1], _R_BLOCK), lambda i: (0, i)),
        ],
        out_specs=pl.BlockSpec(memory_space=pl.ANY),
        out_shape=jax.ShapeDtypeStruct((vocab, batch), jnp.float32),
        scratch_shapes=[
            pltpu.VMEM((2, _R_BLOCK, batch), jnp.float32),
            pltpu.SemaphoreType.DMA((2, _N_CHUNK)),
        ],
    )(embed, wt)


def kernel(word, table, W, b):
    del b
    word = word.astype(jnp.int32)
    vocab, embed_dim = table.shape
    table2 = table.reshape(vocab // 2, 2 * embed_dim)
    pair = _sc_gather(table2, word >> 1)
    embed = jnp.where(
        (word & 1)[:, None] == 1, pair[:, embed_dim:], pair[:, :embed_dim]
    )
    out_t = _proj_softmax_t(embed, W.T)
    return out_t.T


# speedup vs baseline: 4.9589x; 1.4523x over previous
import functools

import jax
import jax.numpy as jnp
from jax.experimental import pallas as pl
from jax.experimental.pallas import tpu as pltpu
from jax.experimental.pallas import tpu_sc as plsc


_GATHER_WINDOW = 128


def _sc_gather(table, word):
    batch, embed_dim = word.shape[0], table.shape[1]
    indices = word.reshape(1, batch)
    mesh = plsc.VectorSubcoreMesh(core_axis_name="core", subcore_axis_name="subcore")

    @pl.kernel(
        out_type=jax.ShapeDtypeStruct((batch, embed_dim), table.dtype),
        mesh=mesh,
    )
    def gather_kernel(table_hbm, idx_hbm, out_hbm):
        def body(idx_vmem, out_vmem):
            pltpu.sync_copy(table_hbm.at[idx_vmem.at[0]], out_vmem)

        pltpu.emit_pipeline(
            body,
            grid=(batch // _GATHER_WINDOW,),
            in_specs=[pl.BlockSpec((1, _GATHER_WINDOW), lambda i: (0, i))],
            out_specs=[pl.BlockSpec((_GATHER_WINDOW, embed_dim), lambda i: (i, 0))],
            core_axis_name="subcore",
            dimension_semantics=(pltpu.PARALLEL,),
        )(idx_hbm, out_hbm)

    return gather_kernel(table, indices)


_R_BLOCK = 2048
_N_CHUNK = 16


def _proj_softmax_t(embed, wt):
    batch = embed.shape[0]
    vocab = wt.shape[1]
    n_steps = pl.cdiv(vocab, _R_BLOCK)
    last = n_steps - 1
    tail = vocab - last * _R_BLOCK

    def chunk_slices(width):
        per = _R_BLOCK // _N_CHUNK
        out = []
        off = 0
        while off < width:
            size = min(per, width - off)
            out.append((off, size))
            off += size
        return out

    per = _R_BLOCK // _N_CHUNK

    def body(embed_ref, w_ref, out_hbm, obuf, sems):
        j = pl.program_id(0)

        def dmas(slot, jj, width):
            return [
                pltpu.make_async_copy(
                    obuf.at[slot, pl.ds(off, size), :],
                    out_hbm.at[pl.ds(jj * _R_BLOCK + off, size), :],
                    sems.at[slot, c],
                )
                for c, (off, size) in enumerate(chunk_slices(width))
            ]

        def run_block(slot, dma_width):
            wb = w_ref[...].astype(jnp.bfloat16)
            eb = embed_ref[...].astype(jnp.bfloat16)
            out_cps = dmas(slot, j, dma_width)
            for c in range(_N_CHUNK):
                off = c * per
                logits_t = jax.lax.dot_general(
                    wb[:, off:off + per], eb,
                    dimension_numbers=(((0,), (1,)), ((), ())),
                    preferred_element_type=jnp.float32,
                )
                row_max = jnp.max(logits_t, axis=1, keepdims=True)
                lse = jnp.log(
                    jnp.sum(jnp.exp(logits_t - row_max), axis=1, keepdims=True))
                obuf[slot, pl.ds(off, per), :] = logits_t - (row_max + lse)
                if c < len(out_cps):
                    out_cps[c].start()

        def step(slot):
            @pl.when(j >= 2)
            def _():
                for cp in dmas(slot, j - 2, _R_BLOCK):
                    cp.wait()

            @pl.when(j != last)
            def _():
                run_block(slot, _R_BLOCK)

            @pl.when(j == last)
            def _():
                run_block(slot, tail)
                for cp in dmas(slot, j, tail):
                    cp.wait()
                for cp in dmas(1 - slot, j - 1, _R_BLOCK):
                    cp.wait()

        @pl.when(j % 2 == 0)
        def _():
            step(0)

        @pl.when(j % 2 == 1)
        def _():
            step(1)

    return pl.pallas_call(
        body,
        grid=(n_steps,),
        in_specs=[
            pl.BlockSpec((batch, embed.shape[1]), lambda i: (0, 0)),
            pl.BlockSpec((embed.shape[1], _R_BLOCK), lambda i: (0, i)),
        ],
        out_specs=pl.BlockSpec(memory_space=pl.ANY),
        out_shape=jax.ShapeDtypeStruct((vocab, batch), jnp.float32),
        scratch_shapes=[
            pltpu.VMEM((2, _R_BLOCK, batch), jnp.float32),
            pltpu.SemaphoreType.DMA((2, _N_CHUNK)),
        ],
    )(embed, wt)


def kernel(word, table, W, b):
    del b
    word = word.astype(jnp.int32)
    vocab, embed_dim = table.shape
    table2 = table.reshape(vocab // 2, 2 * embed_dim)
    pair = _sc_gather(table2, word >> 1)
    embed = jnp.where(
        (word & 1)[:, None] == 1, pair[:, embed_dim:], pair[:, :embed_dim]
    )
    out_t = _proj_softmax_t(embed, W.T)
    return out_t.T


# speedup vs baseline: 5.0104x; 1.0104x over previous
import functools

import jax
import jax.numpy as jnp
from jax.experimental import pallas as pl
from jax.experimental.pallas import tpu as pltpu
from jax.experimental.pallas import tpu_sc as plsc


_GATHER_WINDOW = 128


def _sc_gather(table, word):
    batch, embed_dim = word.shape[0], table.shape[1]
    indices = word.reshape(1, batch)
    mesh = plsc.VectorSubcoreMesh(core_axis_name="core", subcore_axis_name="subcore")

    @pl.kernel(
        out_type=jax.ShapeDtypeStruct((batch, embed_dim), table.dtype),
        mesh=mesh,
    )
    def gather_kernel(table_hbm, idx_hbm, out_hbm):
        def body(idx_vmem, out_vmem):
            pltpu.sync_copy(table_hbm.at[idx_vmem.at[0]], out_vmem)

        pltpu.emit_pipeline(
            body,
            grid=(batch // _GATHER_WINDOW,),
            in_specs=[pl.BlockSpec((1, _GATHER_WINDOW), lambda i: (0, i))],
            out_specs=[pl.BlockSpec((_GATHER_WINDOW, embed_dim), lambda i: (i, 0))],
            core_axis_name="subcore",
            dimension_semantics=(pltpu.PARALLEL,),
        )(idx_hbm, out_hbm)

    return gather_kernel(table, indices)


_R_BLOCK = 4096
_N_CHUNK = 32


def _proj_softmax_t(embed, wt):
    batch = embed.shape[0]
    vocab = wt.shape[1]
    n_steps = pl.cdiv(vocab, _R_BLOCK)
    last = n_steps - 1
    tail = vocab - last * _R_BLOCK

    def chunk_slices(width):
        per = _R_BLOCK // _N_CHUNK
        out = []
        off = 0
        while off < width:
            size = min(per, width - off)
            out.append((off, size))
            off += size
        return out

    per = _R_BLOCK // _N_CHUNK

    def body(embed_ref, w_ref, out_hbm, obuf, sems):
        j = pl.program_id(0)

        def dmas(slot, jj, width):
            return [
                pltpu.make_async_copy(
                    obuf.at[slot, pl.ds(off, size), :],
                    out_hbm.at[pl.ds(jj * _R_BLOCK + off, size), :],
                    sems.at[slot, c],
                )
                for c, (off, size) in enumerate(chunk_slices(width))
            ]

        def run_block(slot, dma_width):
            wb = w_ref[...].astype(jnp.bfloat16)
            eb = embed_ref[...].astype(jnp.bfloat16)
            out_cps = dmas(slot, j, dma_width)
            for c in range(_N_CHUNK):
                off = c * per
                logits_t = jax.lax.dot_general(
                    wb[:, off:off + per], eb,
                    dimension_numbers=(((0,), (1,)), ((), ())),
                    preferred_element_type=jnp.float32,
                )
                row_max = jnp.max(logits_t, axis=1, keepdims=True)
                lse = jnp.log(
                    jnp.sum(jnp.exp(logits_t - row_max), axis=1, keepdims=True))
                obuf[slot, pl.ds(off, per), :] = logits_t - (row_max + lse)
                if c < len(out_cps):
                    out_cps[c].start()

        def step(slot):
            @pl.when(j >= 2)
            def _():
                for cp in dmas(slot, j - 2, _R_BLOCK):
                    cp.wait()

            @pl.when(j != last)
            def _():
                run_block(slot, _R_BLOCK)

            @pl.when(j == last)
            def _():
                run_block(slot, tail)
                for cp in dmas(slot, j, tail):
                    cp.wait()
                for cp in dmas(1 - slot, j - 1, _R_BLOCK):
                    cp.wait()

        @pl.when(j % 2 == 0)
        def _():
            step(0)

        @pl.when(j % 2 == 1)
        def _():
            step(1)

    return pl.pallas_call(
        body,
        grid=(n_steps,),
        in_specs=[
            pl.BlockSpec((batch, embed.shape[1]), lambda i: (0, 0)),
            pl.BlockSpec((embed.shape[1], _R_BLOCK), lambda i: (0, i)),
        ],
        out_specs=pl.BlockSpec(memory_space=pl.ANY),
        out_shape=jax.ShapeDtypeStruct((vocab, batch), jnp.float32),
        scratch_shapes=[
            pltpu.VMEM((2, _R_BLOCK, batch), jnp.float32),
            pltpu.SemaphoreType.DMA((2, _N_CHUNK)),
        ],
    )(embed, wt)


def kernel(word, table, W, b):
    del b
    word = word.astype(jnp.int32)
    vocab, embed_dim = table.shape
    table2 = table.reshape(vocab // 2, 2 * embed_dim)
    pair = _sc_gather(table2, word >> 1)
    embed = jnp.where(
        (word & 1)[:, None] == 1, pair[:, embed_dim:], pair[:, :embed_dim]
    )
    out_t = _proj_softmax_t(embed, W.T)
    return out_t.T


# speedup vs baseline: 5.0335x; 1.0046x over previous
import functools

import jax
import jax.numpy as jnp
from jax.experimental import pallas as pl
from jax.experimental.pallas import tpu as pltpu
from jax.experimental.pallas import tpu_sc as plsc


_GATHER_WINDOW = 128


def _sc_gather(table, word):
    batch, embed_dim = word.shape[0], table.shape[1]
    indices = word.reshape(1, batch)
    mesh = plsc.VectorSubcoreMesh(core_axis_name="core", subcore_axis_name="subcore")

    @pl.kernel(
        out_type=jax.ShapeDtypeStruct((batch, embed_dim), table.dtype),
        mesh=mesh,
        compiler_params=pltpu.CompilerParams(use_tc_tiling_on_sc=False),
    )
    def gather_kernel(table_hbm, idx_hbm, out_hbm):
        def body(idx_vmem, out_vmem):
            pltpu.sync_copy(table_hbm.at[idx_vmem.at[0]], out_vmem)

        pltpu.emit_pipeline(
            body,
            grid=(batch // _GATHER_WINDOW,),
            in_specs=[pl.BlockSpec((1, _GATHER_WINDOW), lambda i: (0, i))],
            out_specs=[pl.BlockSpec((_GATHER_WINDOW, embed_dim), lambda i: (i, 0))],
            core_axis_name="subcore",
            dimension_semantics=(pltpu.PARALLEL,),
        )(idx_hbm, out_hbm)

    return gather_kernel(table, indices)


_R_BLOCK = 4096
_N_CHUNK = 32


def _proj_softmax_t(embed, wt):
    batch = embed.shape[0]
    vocab = wt.shape[1]
    n_steps = pl.cdiv(vocab, _R_BLOCK)
    last = n_steps - 1
    tail = vocab - last * _R_BLOCK

    def chunk_slices(width):
        per = _R_BLOCK // _N_CHUNK
        out = []
        off = 0
        while off < width:
            size = min(per, width - off)
            out.append((off, size))
            off += size
        return out

    per = _R_BLOCK // _N_CHUNK

    def body(embed_ref, w_ref, out_hbm, obuf, sems):
        j = pl.program_id(0)

        def dmas(slot, jj, width):
            return [
                pltpu.make_async_copy(
                    obuf.at[slot, pl.ds(off, size), :],
                    out_hbm.at[pl.ds(jj * _R_BLOCK + off, size), :],
                    sems.at[slot, c],
                )
                for c, (off, size) in enumerate(chunk_slices(width))
            ]

        def run_block(slot, dma_width):
            wb = w_ref[...].astype(jnp.bfloat16)
            eb = embed_ref[...].astype(jnp.bfloat16)
            out_cps = dmas(slot, j, dma_width)
            for c in range(_N_CHUNK):
                off = c * per
                logits_t = jax.lax.dot_general(
                    wb[:, off:off + per], eb,
                    dimension_numbers=(((0,), (1,)), ((), ())),
                    preferred_element_type=jnp.float32,
                )
                row_max = jnp.max(logits_t, axis=1, keepdims=True)
                lse = jnp.log(
                    jnp.sum(jnp.exp(logits_t - row_max), axis=1, keepdims=True))
                obuf[slot, pl.ds(off, per), :] = logits_t - (row_max + lse)
                if c < len(out_cps):
                    out_cps[c].start()

        def step(slot):
            @pl.when(j >= 2)
            def _():
                for cp in dmas(slot, j - 2, _R_BLOCK):
                    cp.wait()

            @pl.when(j != last)
            def _():
                run_block(slot, _R_BLOCK)

            @pl.when(j == last)
            def _():
                run_block(slot, tail)
                for cp in dmas(slot, j, tail):
                    cp.wait()
                for cp in dmas(1 - slot, j - 1, _R_BLOCK):
                    cp.wait()

        @pl.when(j % 2 == 0)
        def _():
            step(0)

        @pl.when(j % 2 == 1)
        def _():
            step(1)

    return pl.pallas_call(
        body,
        grid=(n_steps,),
        in_specs=[
            pl.BlockSpec((batch, embed.shape[1]), lambda i: (0, 0)),
            pl.BlockSpec((embed.shape[1], _R_BLOCK), lambda i: (0, i)),
        ],
        out_specs=pl.BlockSpec(memory_space=pl.ANY),
        out_shape=jax.ShapeDtypeStruct((vocab, batch), jnp.float32),
        scratch_shapes=[
            pltpu.VMEM((2, _R_BLOCK, batch), jnp.float32),
            pltpu.SemaphoreType.DMA((2, _N_CHUNK)),
        ],
    )(embed, wt)


def kernel(word, table, W, b):
    del b
    word = word.astype(jnp.int32)
    vocab, embed_dim = table.shape
    embed = _sc_gather(table, word)
    out_t = _proj_softmax_t(embed, W.T)
    return out_t.T
